# baseline (speedup 1.0000x reference)
import functools

import jax
import jax.numpy as jnp
from jax import lax
from jax.experimental import pallas as pl
from jax.experimental.pallas import tpu as pltpu
from jax.experimental.pallas import tpu_sc as plsc

NC = 2
NS = 16
NW = NC * NS
NB = 128
L = 4
TC_BLK = 512


def _sc_pool_kernel(n_rows_pad, n_idx_rows, emb):
    rpw = n_rows_pad // NW
    n_blocks = rpw // NB
    n_pairs = n_blocks // 2
    out_rows_per_blk = NB * emb // 128
    mesh = plsc.VectorSubcoreMesh(
        core_axis_name="c", subcore_axis_name="s", num_cores=NC, num_subcores=NS
    )

    @functools.partial(
        pl.kernel,
        out_type=jax.ShapeDtypeStruct((n_rows_pad * emb // 128, 128), jnp.float32),
        mesh=mesh,
        scratch_types=[
            pltpu.VMEM((L, 128), jnp.int32),
            pltpu.VMEM((L, 128), jnp.int32),
            pltpu.VMEM((L, NB, emb), jnp.float32),
            pltpu.VMEM((L, NB, emb), jnp.float32),
            pltpu.VMEM((out_rows_per_blk, 128), jnp.float32),
            pltpu.VMEM((out_rows_per_blk, 128), jnp.float32),
            pltpu.SemaphoreType.DMA,
            pltpu.SemaphoreType.DMA,
        ],
        compiler_params=pltpu.CompilerParams(use_tc_tiling_on_sc=False),
    )
    def sc_kernel(idx_hbm, table_hbm, pooled_hbm,
                  idx_a, idx_b, r_a, r_b, out_a, out_b, sem_a, sem_b):
        wid = lax.axis_index("s") * NC + lax.axis_index("c")
        base = wid * rpw

        def fetch(idx_v, r_v, sem, b):
            irow = (base + b * NB) * L // 128
            pltpu.sync_copy(idx_hbm.at[pl.ds(irow, L)], idx_v)
            for l in range(L):
                pltpu.async_copy(table_hbm.at[idx_v.at[l]], r_v.at[l], sem)

        def drain(idx_v, r_v, sem):
            for l in range(L):
                pltpu.make_async_copy(table_hbm.at[idx_v.at[l]], r_v.at[l], sem).wait()

        def pool_write(r_v, out_v, b):
            for l in range(L):
                def body(jj, c, l=l):
                    for q in range(4):
                        for h in range(emb // 16):
                            s = pl.ds(h * 16, 16)
                            g = 16 * jj + 4 * q
                            out_v[8 * l + jj, pl.ds(32 * q + 16 * h, 16)] = (
                                (r_v[l, g, s] + r_v[l, g + 1, s])
                                + (r_v[l, g + 2, s] + r_v[l, g + 3, s]))
                    return c

                lax.fori_loop(0, 8, body, 0, unroll=2)
            orow = (base + b * NB) * emb // 128
            pltpu.sync_copy(out_v, pooled_hbm.at[pl.ds(orow, out_rows_per_blk)])

        fetch(idx_a, r_a, sem_a, 0)

        def pair(pb, c):
            b0 = pb * 2
            fetch(idx_b, r_b, sem_b, b0 + 1)
            drain(idx_a, r_a, sem_a)
            pool_write(r_a, out_a, b0)
            fetch(idx_a, r_a, sem_a, b0 + 2)
            drain(idx_b, r_b, sem_b)
            pool_write(r_b, out_b, b0 + 1)
            return c

        lax.fori_loop(0, n_pairs, pair, 0)
        drain(idx_a, r_a, sem_a)

    return sc_kernel


def _tc_proj_kernel(x_ref, wc_ref, wf_ref, oc_ref, of_ref, *, n_cells_blocks):
    pid = pl.program_id(0)

    @pl.when(pid < n_cells_blocks)
    def _():
        oc_ref[...] = jnp.dot(x_ref[...], wc_ref[...],
                              preferred_element_type=jnp.float32)

    @pl.when(pid >= n_cells_blocks)
    def _():
        of_ref[...] = jnp.dot(x_ref[...], wf_ref[...],
                              preferred_element_type=jnp.float32)


def _block_diag4(w_t):
    emb, hid = w_t.shape
    eye = jnp.eye(4, dtype=w_t.dtype)
    return (eye[:, None, :, None] * w_t[None, :, None, :]).reshape(4 * emb, 4 * hid)


def kernel(fields, cells, W_emb, W_fields, W_cells):
    B, K, Lf = fields.shape
    _, N, _, Lc = cells.shape
    assert Lf == L and Lc == L
    emb = W_emb.shape[1]
    hid = W_fields.shape[0]

    r_cells = B * N * K
    r_fields = B * K
    r = r_cells + r_fields
    unit = NW * NB * 2
    r_pad = ((r + unit - 1) // unit) * unit

    n_idx_rows = r_pad * L // 128 + 8
    idx = jnp.concatenate(
        [cells.reshape(-1), fields.reshape(-1)]).astype(jnp.int32)
    idx = jnp.pad(idx, (0, n_idx_rows * 128 - r * L)).reshape(n_idx_rows, 128)

    pooled = _sc_pool_kernel(r_pad, n_idx_rows, emb)(idx, W_emb)

    wc_big = _block_diag4((W_cells.T * (1.0 / L)).astype(jnp.float32))
    wf_big = _block_diag4((W_fields.T * (1.0 / L)).astype(jnp.float32))

    wide = 128 // emb
    assert (r_cells // wide) % TC_BLK == 0 and (r_fields // wide) % TC_BLK == 0
    n_cells_blocks = r_cells // wide // TC_BLK
    n_blocks = r // wide // TC_BLK
    kop = functools.partial(_tc_proj_kernel, n_cells_blocks=n_cells_blocks)

    out_c, out_f = pl.pallas_call(
        kop,
        grid=(n_blocks,),
        in_specs=[
            pl.BlockSpec((TC_BLK, 128), lambda b: (b, 0)),
            pl.BlockSpec((128, wide * hid), lambda b: (0, 0)),
            pl.BlockSpec((128, wide * hid), lambda b: (0, 0)),
        ],
        out_specs=[
            pl.BlockSpec((TC_BLK, wide * hid),
                         lambda b: (jnp.minimum(b, n_cells_blocks - 1), 0)),
            pl.BlockSpec((TC_BLK, wide * hid),
                         lambda b: (jnp.maximum(b - n_cells_blocks, 0), 0)),
        ],
        out_shape=[
            jax.ShapeDtypeStruct((r_cells // wide, wide * hid), jnp.float32),
            jax.ShapeDtypeStruct((r_fields // wide, wide * hid), jnp.float32),
        ],
        compiler_params=pltpu.CompilerParams(
            dimension_semantics=("arbitrary",),
        ),
    )(pooled, wc_big, wf_big)

    db_cells_out = out_c.reshape(B, N, K, hid)
    db_fields_out = out_f.reshape(B, K, hid)
    return (db_fields_out, db_cells_out)


# speedup vs baseline: 1.6565x; 1.6565x over previous
import functools

import jax
import jax.numpy as jnp
from jax import lax
from jax.experimental import pallas as pl
from jax.experimental.pallas import tpu as pltpu
from jax.experimental.pallas import tpu_sc as plsc

NC = 2
NS = 16
NW = NC * NS
BC = 128
L = 4
TC_BLK = 512


def _sc_pool_kernel(n_grp_pad, n_idx_grp, chunks, emb):
    units = n_grp_pad * chunks
    upw = units // NW
    n_pairs = upw // 2
    owpu = BC * emb // 128
    mesh = plsc.VectorSubcoreMesh(
        core_axis_name="c", subcore_axis_name="s", num_cores=NC, num_subcores=NS
    )

    @functools.partial(
        pl.kernel,
        out_type=jax.ShapeDtypeStruct((units * owpu, 128), jnp.float32),
        mesh=mesh,
        scratch_types=[
            pltpu.VMEM((L, BC), jnp.int32),
            pltpu.VMEM((L, BC), jnp.int32),
            pltpu.VMEM((L, BC, emb), jnp.float32),
            pltpu.VMEM((L, BC, emb), jnp.float32),
            pltpu.VMEM((owpu, 128), jnp.float32),
            pltpu.VMEM((owpu, 128), jnp.float32),
            pltpu.SemaphoreType.DMA,
            pltpu.SemaphoreType.DMA,
        ],
        compiler_params=pltpu.CompilerParams(use_tc_tiling_on_sc=False),
    )
    def sc_kernel(idx_hbm, table_hbm, pooled_hbm,
                  idx_a, idx_b, r_a, r_b, out_a, out_b, sem_a, sem_b):
        wid = lax.axis_index("s") * NC + lax.axis_index("c")
        u_base = wid * upw

        def fetch(idx_v, r_v, sem, t):
            u = u_base + t
            g = u // chunks
            c = lax.rem(u, chunks)
            for l in range(L):
                pltpu.sync_copy(idx_hbm.at[g, l, c], idx_v.at[l])
            for l in range(L):
                pltpu.async_copy(table_hbm.at[idx_v.at[l]], r_v.at[l], sem)

        def drain(idx_v, r_v, sem):
            for l in range(L):
                pltpu.make_async_copy(table_hbm.at[idx_v.at[l]], r_v.at[l], sem).wait()

        def pool_write(r_v, out_v, t):
            def body(jw, carry):
                for q in range(4):
                    for h in range(emb // 16):
                        s = pl.ds(h * 16, 16)
                        j = 4 * jw + q
                        out_v[jw, pl.ds(32 * q + 16 * h, 16)] = (
                            (r_v[0, j, s] + r_v[1, j, s])
                            + (r_v[2, j, s] + r_v[3, j, s]))
                return carry

            lax.fori_loop(0, owpu, body, 0, unroll=2)
            u = u_base + t
            pltpu.sync_copy(out_v, pooled_hbm.at[pl.ds(u * owpu, owpu)])

        fetch(idx_a, r_a, sem_a, 0)

        def pair(pb, carry):
            t0 = pb * 2
            fetch(idx_b, r_b, sem_b, t0 + 1)
            drain(idx_a, r_a, sem_a)
            pool_write(r_a, out_a, t0)
            fetch(idx_a, r_a, sem_a, t0 + 2)
            drain(idx_b, r_b, sem_b)
            pool_write(r_b, out_b, t0 + 1)
            return carry

        lax.fori_loop(0, n_pairs, pair, 0)
        drain(idx_a, r_a, sem_a)

    return sc_kernel


def _tc_proj_kernel(x_ref, wc_ref, wf_ref, oc_ref, of_ref, *, n_cells_blocks):
    pid = pl.program_id(0)

    @pl.when(pid < n_cells_blocks)
    def _():
        oc_ref[...] = jnp.dot(x_ref[...], wc_ref[...],
                              preferred_element_type=jnp.float32)

    @pl.when(pid >= n_cells_blocks)
    def _():
        of_ref[...] = jnp.dot(x_ref[...], wf_ref[...],
                              preferred_element_type=jnp.float32)


def _block_diag4(w_t):
    emb, hid = w_t.shape
    eye = jnp.eye(4, dtype=w_t.dtype)
    return (eye[:, None, :, None] * w_t[None, :, None, :]).reshape(4 * emb, 4 * hid)


def kernel(fields, cells, W_emb, W_fields, W_cells):
    B, K, Lf = fields.shape
    _, N, _, Lc = cells.shape
    assert Lf == L and Lc == L and B % BC == 0
    emb = W_emb.shape[1]
    hid = W_fields.shape[0]
    chunks = B // BC

    g_cells = N * K
    g_fields = K
    g_real = g_cells + g_fields
    g_unit = (2 * NW) // chunks if (2 * NW) % chunks == 0 else 2 * NW
    g_pad = ((g_real + g_unit - 1) // g_unit) * g_unit
    n_idx_grp = g_pad + 8

    cells_t = cells.astype(jnp.int32).transpose(1, 2, 3, 0)
    fields_t = fields.astype(jnp.int32).transpose(1, 2, 0)
    idx3 = jnp.concatenate([
        cells_t.reshape(g_cells, L, chunks, BC),
        fields_t.reshape(g_fields, L, chunks, BC),
        jnp.zeros((n_idx_grp - g_real, L, chunks, BC), jnp.int32),
    ], axis=0)

    pooled = _sc_pool_kernel(g_pad, n_idx_grp, chunks, emb)(idx3, W_emb)

    wc_big = _block_diag4((W_cells.T * (1.0 / L)).astype(jnp.float32))
    wf_big = _block_diag4((W_fields.T * (1.0 / L)).astype(jnp.float32))

    wide = 128 // emb
    assert (g_cells * B // wide) % TC_BLK == 0
    assert (g_fields * B // wide) % TC_BLK == 0
    n_cells_blocks = g_cells * B // wide // TC_BLK
    n_blocks = g_real * B // wide // TC_BLK
    kop = functools.partial(_tc_proj_kernel, n_cells_blocks=n_cells_blocks)

    out_c, out_f = pl.pallas_call(
        kop,
        grid=(n_blocks,),
        in_specs=[
            pl.BlockSpec((TC_BLK, 128), lambda b: (b, 0)),
            pl.BlockSpec((128, wide * hid), lambda b: (0, 0)),
            pl.BlockSpec((128, wide * hid), lambda b: (0, 0)),
        ],
        out_specs=[
            pl.BlockSpec((TC_BLK, wide * hid),
                         lambda b: (jnp.minimum(b, n_cells_blocks - 1), 0)),
            pl.BlockSpec((TC_BLK, wide * hid),
                         lambda b: (jnp.maximum(b - n_cells_blocks, 0), 0)),
        ],
        out_shape=[
            jax.ShapeDtypeStruct((g_cells * B // wide, wide * hid), jnp.float32),
            jax.ShapeDtypeStruct((g_fields * B // wide, wide * hid), jnp.float32),
        ],
        compiler_params=pltpu.CompilerParams(
            dimension_semantics=("arbitrary",),
        ),
    )(pooled, wc_big, wf_big)

    db_cells_out = out_c.reshape(N, K, B, hid).transpose(2, 0, 1, 3)
    db_fields_out = out_f.reshape(K, B, hid).transpose(1, 0, 2)
    return (db_fields_out, db_cells_out)
